# baseline (speedup 1.0000x reference)
import functools

import jax
import jax.numpy as jnp
from jax import lax
from jax.experimental import pallas as pl
from jax.experimental.pallas import tpu as pltpu
from jax.experimental.pallas import tpu_sc as plsc

N = 10000
E = 320000
D = 128
DH = 64
K = 10
ALPHA = 0.1

NP = 10240
RT = NP // 16
RC = RT // 128
EP = 327680
ECH = EP // (16 * 128)

f32 = jnp.float32
i32 = jnp.int32


def _mlp_body(x_ref, w1_ref, b1_ref, w2_ref, b2_ref, dgo_ref, dgi_ref,
              h1_ref, sa_ref, sb_ref, za_ref, zb_ref, aa_ref, ab_ref,
              cc_ref, ci_ref):
    x = x_ref[...]
    dn = (((1,), (1,)), ((), ()))
    h1 = lax.dot_general(x, w1_ref[...], dn,
                         preferred_element_type=f32,
                         precision=lax.Precision.HIGHEST) + b1_ref[...]
    h1_ref[...] = h1
    h = lax.dot_general(jax.nn.relu(h1), w2_ref[...], dn,
                        preferred_element_type=f32,
                        precision=lax.Precision.HIGHEST) + b2_ref[...]
    no = lax.rsqrt(jnp.maximum(dgo_ref[...][:, :1], 1.0))
    ni = lax.rsqrt(jnp.maximum(dgi_ref[...][:, :1], 1.0))
    s0 = no * h
    z0 = ALPHA * s0
    az = ALPHA * h
    sa_ref[...] = s0[:, :DH]
    sb_ref[...] = s0[:, DH:]
    za_ref[...] = z0[:, :DH]
    zb_ref[...] = z0[:, DH:]
    aa_ref[...] = az[:, :DH]
    ab_ref[...] = az[:, DH:]
    cc_ref[...] = jnp.broadcast_to((1.0 - ALPHA) * no * ni, cc_ref.shape)
    ci_ref[...] = jnp.broadcast_to((1.0 - ALPHA) * ni, ci_ref.shape)


def _mlp(featsp, W1, b1r, W2, b2r, dgo, dgi):
    R = 1024
    grid = (NP // R,)
    row = lambda i: (i, 0)
    fixed = lambda i: (0, 0)
    out_shapes = (
        jax.ShapeDtypeStruct((NP, D), f32),
        jax.ShapeDtypeStruct((NP, DH), f32),
        jax.ShapeDtypeStruct((NP, DH), f32),
        jax.ShapeDtypeStruct((NP, DH), f32),
        jax.ShapeDtypeStruct((NP, DH), f32),
        jax.ShapeDtypeStruct((NP, DH), f32),
        jax.ShapeDtypeStruct((NP, DH), f32),
        jax.ShapeDtypeStruct((NP, 16), f32),
        jax.ShapeDtypeStruct((NP, 16), f32),
    )
    return pl.pallas_call(
        _mlp_body,
        grid=grid,
        in_specs=[
            pl.BlockSpec((R, D), row),
            pl.BlockSpec((D, D), fixed),
            pl.BlockSpec((1, D), fixed),
            pl.BlockSpec((D, D), fixed),
            pl.BlockSpec((1, D), fixed),
            pl.BlockSpec((R, 16), row),
            pl.BlockSpec((R, 16), row),
        ],
        out_specs=(
            pl.BlockSpec((R, D), row),
            pl.BlockSpec((R, DH), row),
            pl.BlockSpec((R, DH), row),
            pl.BlockSpec((R, DH), row),
            pl.BlockSpec((R, DH), row),
            pl.BlockSpec((R, DH), row),
            pl.BlockSpec((R, DH), row),
            pl.BlockSpec((R, 16), row),
            pl.BlockSpec((R, 16), row),
        ),
        out_shape=out_shapes,
    )(featsp, W1, b1r, W2, b2r, dgo, dgi)


def _deg_body(edf, ones_h, zer_h, out, idx_v, ones_v, buf_v, deg_sp, sem):
    c = lax.axis_index("c")
    s = lax.axis_index("s")
    rb = s * RT
    NCH = EP // 128
    pltpu.sync_copy(edf.at[pl.ds(c * NCH + s * ECH, ECH)], idx_v)
    pltpu.sync_copy(ones_h, ones_v)
    pltpu.sync_copy(zer_h, buf_v)
    pltpu.sync_copy(buf_v, deg_sp.at[pl.ds(rb, RT)])
    plsc.subcore_barrier()
    def chunk(j, carry):
        pltpu.sync_copy(ones_v, deg_sp.at[idx_v.at[j]], add=True)
        return carry
    lax.fori_loop(0, ECH, chunk, 0)
    plsc.subcore_barrier()
    pltpu.sync_copy(deg_sp.at[pl.ds(rb, RT)], buf_v)
    pltpu.sync_copy(buf_v, out.at[pl.ds(c * NP + rb, RT)])


def _degrees(edf, ones_h, zer_h):
    mesh = plsc.VectorSubcoreMesh(core_axis_name="c", subcore_axis_name="s")
    return pl.kernel(
        _deg_body,
        out_type=jax.ShapeDtypeStruct((2 * NP, 16), f32),
        mesh=mesh,
        compiler_params=pltpu.CompilerParams(needs_layout_passes=False,
                                             use_tc_tiling_on_sc=False),
        scratch_types=[
            pltpu.VMEM((ECH, 128), i32),
            pltpu.VMEM((128, 16), f32),
            pltpu.VMEM((RT, 16), f32),
            pltpu.VMEM_SHARED((NP, 16), f32),
            pltpu.SemaphoreType.DMA,
        ],
    )(edf, ones_h, zer_h)


MSGN = 4
NQ = ECH // MSGN


def _diff_body(S2, Z2, AZ2, cc_h, ci_h, srcp2, dstp, zer_h, out, s_h,
               sidx_v, didx_v, m0, m1, m2, m3, cc_v, ci_v, zero_v,
               agg_sp, g0, g1, g2, g3, t0, t1, t2, t3):
    msg = (m0, m1, m2, m3)
    gsem = (g0, g1, g2, g3)
    ssem = (t0, t1, t2, t3)
    c = lax.axis_index("c")
    s = lax.axis_index("s")
    rb = s * RT
    gb = c * NP + rb
    NCH = EP // 128

    pltpu.sync_copy(srcp2.at[pl.ds(c * NCH + s * ECH, ECH)], sidx_v)
    pltpu.sync_copy(dstp.at[pl.ds(s * ECH, ECH)], didx_v)
    pltpu.sync_copy(cc_h.at[pl.ds(rb, RT)], cc_v)
    pltpu.sync_copy(ci_h.at[pl.ds(rb, RT)], ci_v)
    pltpu.sync_copy(zer_h, zero_v)
    for k in range(RC):
        pltpu.sync_copy(zero_v, agg_sp.at[pl.ds(rb + k * 128, 128)])
    plsc.subcore_barrier()

    def combine_row(r, k, final):
        lr = k * 128 + r
        idx = jnp.full((16,), lr, dtype=i32)
        cv = plsc.load_gather(ci_v if final else cc_v, [idx])
        for q in range(DH // 16):
            sl = pl.ds(q * 16, 16)
            msg[0][r, sl] = cv * msg[0][r, sl] + msg[1][r, sl]

    for it in range(K):
        final = it == K - 1
        tbl = S2 if it == 0 else s_h

        for b in range(MSGN):
            pltpu.async_copy(tbl.at[sidx_v.at[b]], msg[b], gsem[b])

        def quad(q, carry):
            hs = []
            for b in range(MSGN):
                pltpu.make_async_copy(zer_h, msg[b], gsem[b]).wait()
                hs.append(pltpu.async_copy(
                    msg[b], agg_sp.at[didx_v.at[q * MSGN + b]], ssem[b],
                    add=True))
            for b in range(MSGN):
                hs[b].wait()

                @pl.when(q < NQ - 1)
                def _(b=b):
                    pltpu.async_copy(tbl.at[sidx_v.at[(q + 1) * MSGN + b]],
                                     msg[b], gsem[b])
            return carry

        lax.fori_loop(0, NQ, quad, 0)
        plsc.subcore_barrier()
        for k in range(RC):
            base = rb + k * 128
            pltpu.sync_copy(agg_sp.at[pl.ds(base, 128)], msg[0])
            bias = AZ2 if final else Z2
            pltpu.sync_copy(bias.at[pl.ds(gb + k * 128, 128)], msg[1])

            def rbody(r, carry, _k=k, _final=final):
                combine_row(r, _k, _final)
                return carry

            lax.fori_loop(0, 128, rbody, 0)
            if final:
                pltpu.sync_copy(msg[0], out.at[pl.ds(gb + k * 128, 128)])
            else:
                pltpu.sync_copy(msg[0], s_h.at[pl.ds(gb + k * 128, 128)])
                pltpu.sync_copy(zero_v, agg_sp.at[pl.ds(base, 128)])
        if not final:
            plsc.subcore_barrier()


def _diffuse(S2, Z2, AZ2, cc1, ci1, srcp2, dstp, zer_h):
    mesh = plsc.VectorSubcoreMesh(core_axis_name="c", subcore_axis_name="s")
    out, _ = pl.kernel(
        _diff_body,
        out_type=(jax.ShapeDtypeStruct((2 * NP, DH), f32),
                  jax.ShapeDtypeStruct((2 * NP, DH), f32)),
        mesh=mesh,
        compiler_params=pltpu.CompilerParams(needs_layout_passes=False,
                                             use_tc_tiling_on_sc=False),
        scratch_types=[
            pltpu.VMEM((ECH, 128), i32),
            pltpu.VMEM((ECH, 128), i32),
            pltpu.VMEM((128, DH), f32),
            pltpu.VMEM((128, DH), f32),
            pltpu.VMEM((128, DH), f32),
            pltpu.VMEM((128, DH), f32),
            pltpu.VMEM((RT,), f32),
            pltpu.VMEM((RT,), f32),
            pltpu.VMEM((128, DH), f32),
            pltpu.VMEM_SHARED((NP, DH), f32),
            pltpu.SemaphoreType.DMA,
            pltpu.SemaphoreType.DMA,
            pltpu.SemaphoreType.DMA,
            pltpu.SemaphoreType.DMA,
            pltpu.SemaphoreType.DMA,
            pltpu.SemaphoreType.DMA,
            pltpu.SemaphoreType.DMA,
            pltpu.SemaphoreType.DMA,
        ],
    )(S2, Z2, AZ2, cc1, ci1, srcp2, dstp, zer_h)
    return out


def kernel(edge_index, feats, W1, b1, W2, b2):
    featsp = jnp.pad(feats, ((0, NP - N), (0, 0)))
    padidx = (N + (jnp.arange(EP - E, dtype=i32) % (NP - N))).astype(i32)
    srcp = jnp.concatenate([edge_index[0], padidx]).reshape(EP // 128, 128)
    dstp = jnp.concatenate([edge_index[1], padidx]).reshape(EP // 128, 128)
    ones_h = jnp.ones((128, 16), f32)
    zer16 = jnp.zeros((RT, 16), f32)
    zer_h = jnp.zeros((128, DH), f32)

    edf = jnp.concatenate([srcp, dstp])
    degs = _degrees(edf, ones_h, zer16)
    h1p, sA, sB, zA, zB, aA, aB, cc16, ci16 = _mlp(
        featsp, W1, b1.reshape(1, D), W2, b2.reshape(1, D),
        degs[:NP], degs[NP:])

    if False:
        no = lax.rsqrt(jnp.maximum(degs[:N, 0], 1.0))
        ni = lax.rsqrt(jnp.maximum(degs[NP:NP + N, 0], 1.0))
        h0 = jnp.concatenate([sA, sB], axis=1)[:N] / no[:, None]
        h = h0
        src = edge_index[0]
        dst = edge_index[1]
        for _ in range(K):
            hs = h * no[:, None]
            agg = jax.ops.segment_sum(jnp.take(hs, src, axis=0), dst,
                                      num_segments=N)
            h = (1.0 - ALPHA) * agg * ni[:, None] + ALPHA * h0
        return (h1p[:N], h)

    S2 = jnp.concatenate([sA, sB], axis=0)
    Z2 = jnp.concatenate([zA, zB], axis=0)
    AZ2 = jnp.concatenate([aA, aB], axis=0)
    srcp2 = jnp.concatenate([srcp, srcp + NP])
    out = _diffuse(S2, Z2, AZ2, cc16[:, 0], ci16[:, 0], srcp2, dstp, zer_h)
    h = jnp.concatenate([out[:N], out[NP:NP + N]], axis=1)
    return (h1p[:N], h)


# speedup vs baseline: 1.1823x; 1.1823x over previous
import functools

import jax
import jax.numpy as jnp
from jax import lax
from jax.experimental import pallas as pl
from jax.experimental.pallas import tpu as pltpu
from jax.experimental.pallas import tpu_sc as plsc

N = 10000
E = 320000
D = 128
DH = 64
K = 10
ALPHA = 0.1

NP = 10240
RT = NP // 16
RC = RT // 128
EP = 327680
ECH = EP // (16 * 128)

f32 = jnp.float32
i32 = jnp.int32


def _mlp_body(x_ref, w1_ref, b1_ref, w2_ref, b2_ref, dgo_ref, dgi_ref,
              h1_ref, sa_ref, sb_ref, za_ref, zb_ref, aa_ref, ab_ref,
              cc_ref, ci_ref):
    x = x_ref[...]
    dn = (((1,), (1,)), ((), ()))
    h1 = lax.dot_general(x, w1_ref[...], dn,
                         preferred_element_type=f32,
                         precision=lax.Precision.HIGHEST) + b1_ref[...]
    h1_ref[...] = h1
    h = lax.dot_general(jax.nn.relu(h1), w2_ref[...], dn,
                        preferred_element_type=f32,
                        precision=lax.Precision.HIGHEST) + b2_ref[...]
    no = lax.rsqrt(jnp.maximum(dgo_ref[...][:, :1], 1.0))
    ni = lax.rsqrt(jnp.maximum(dgi_ref[...][:, :1], 1.0))
    s0 = no * h
    z0 = ALPHA * s0
    az = ALPHA * h
    sa_ref[...] = s0[:, :DH]
    sb_ref[...] = s0[:, DH:]
    za_ref[...] = z0[:, :DH]
    zb_ref[...] = z0[:, DH:]
    aa_ref[...] = az[:, :DH]
    ab_ref[...] = az[:, DH:]
    cc_ref[...] = jnp.broadcast_to((1.0 - ALPHA) * no * ni, cc_ref.shape)
    ci_ref[...] = jnp.broadcast_to((1.0 - ALPHA) * ni, ci_ref.shape)


def _mlp(featsp, W1, b1r, W2, b2r, dgo, dgi):
    R = 1024
    grid = (NP // R,)
    row = lambda i: (i, 0)
    fixed = lambda i: (0, 0)
    out_shapes = (
        jax.ShapeDtypeStruct((NP, D), f32),
        jax.ShapeDtypeStruct((NP, DH), f32),
        jax.ShapeDtypeStruct((NP, DH), f32),
        jax.ShapeDtypeStruct((NP, DH), f32),
        jax.ShapeDtypeStruct((NP, DH), f32),
        jax.ShapeDtypeStruct((NP, DH), f32),
        jax.ShapeDtypeStruct((NP, DH), f32),
        jax.ShapeDtypeStruct((NP, DH), f32),
        jax.ShapeDtypeStruct((NP, DH), f32),
    )
    return pl.pallas_call(
        _mlp_body,
        grid=grid,
        in_specs=[
            pl.BlockSpec((R, D), row),
            pl.BlockSpec((D, D), fixed),
            pl.BlockSpec((1, D), fixed),
            pl.BlockSpec((D, D), fixed),
            pl.BlockSpec((1, D), fixed),
            pl.BlockSpec((R, 16), row),
            pl.BlockSpec((R, 16), row),
        ],
        out_specs=(
            pl.BlockSpec((R, D), row),
            pl.BlockSpec((R, DH), row),
            pl.BlockSpec((R, DH), row),
            pl.BlockSpec((R, DH), row),
            pl.BlockSpec((R, DH), row),
            pl.BlockSpec((R, DH), row),
            pl.BlockSpec((R, DH), row),
            pl.BlockSpec((R, DH), row),
            pl.BlockSpec((R, DH), row),
        ),
        out_shape=out_shapes,
    )(featsp, W1, b1r, W2, b2r, dgo, dgi)


def _deg_body(edf, ones_h, zer_h, out, idx_v, ones_v, buf_v, deg_sp, sem):
    c = lax.axis_index("c")
    s = lax.axis_index("s")
    rb = s * RT
    NCH = EP // 128
    pltpu.sync_copy(edf.at[pl.ds(c * NCH + s * ECH, ECH)], idx_v)
    pltpu.sync_copy(ones_h, ones_v)
    pltpu.sync_copy(zer_h, buf_v)
    pltpu.sync_copy(buf_v, deg_sp.at[pl.ds(rb, RT)])
    plsc.subcore_barrier()
    def chunk(j, carry):
        pltpu.sync_copy(ones_v, deg_sp.at[idx_v.at[j]], add=True)
        return carry
    lax.fori_loop(0, ECH, chunk, 0)
    plsc.subcore_barrier()
    pltpu.sync_copy(deg_sp.at[pl.ds(rb, RT)], buf_v)
    pltpu.sync_copy(buf_v, out.at[pl.ds(c * NP + rb, RT)])


def _degrees(edf, ones_h, zer_h):
    mesh = plsc.VectorSubcoreMesh(core_axis_name="c", subcore_axis_name="s")
    return pl.kernel(
        _deg_body,
        out_type=jax.ShapeDtypeStruct((2 * NP, 16), f32),
        mesh=mesh,
        compiler_params=pltpu.CompilerParams(needs_layout_passes=False,
                                             use_tc_tiling_on_sc=False),
        scratch_types=[
            pltpu.VMEM((ECH, 128), i32),
            pltpu.VMEM((128, 16), f32),
            pltpu.VMEM((RT, 16), f32),
            pltpu.VMEM_SHARED((NP, 16), f32),
            pltpu.SemaphoreType.DMA,
        ],
    )(edf, ones_h, zer_h)


MSGN = 4
NQ = ECH // MSGN


def _diff_body(S2, Z2, AZ2, cc_h, ci_h, srcp2, dstp, zer_h, out, s_h,
               sidx_v, didx_v, m0, m1, m2, m3, zero_v,
               agg_sp, g0, g1, g2, g3):
    msg = (m0, m1, m2, m3)
    gsem = (g0, g1, g2, g3)
    c = lax.axis_index("c")
    s = lax.axis_index("s")
    rb = s * RT
    gb = c * NP + rb
    NCH = EP // 128

    pltpu.sync_copy(srcp2.at[pl.ds(c * NCH + s * ECH, ECH)], sidx_v)
    pltpu.sync_copy(dstp.at[pl.ds(s * ECH, ECH)], didx_v)
    pltpu.sync_copy(zer_h, zero_v)
    for k in range(RC):
        pltpu.sync_copy(zero_v, agg_sp.at[pl.ds(rb + k * 128, 128)])
    plsc.subcore_barrier()

    def combine_row(r, carry):
        for q in range(DH // 16):
            sl = pl.ds(q * 16, 16)
            msg[0][r, sl] = msg[2][r, sl] * msg[0][r, sl] + msg[1][r, sl]
        return carry

    for it in range(K):
        final = it == K - 1
        tbl = S2 if it == 0 else s_h

        for b in range(MSGN):
            pltpu.async_copy(tbl.at[sidx_v.at[b]], msg[b], gsem[b])

        def quad(q, carry):
            for b in range(MSGN):
                pltpu.make_async_copy(zer_h, msg[b], gsem[b]).wait()
                pltpu.sync_copy(msg[b], agg_sp.at[didx_v.at[q * MSGN + b]],
                                add=True)

                @pl.when(q < NQ - 1)
                def _(b=b):
                    pltpu.async_copy(tbl.at[sidx_v.at[(q + 1) * MSGN + b]],
                                     msg[b], gsem[b])
            return carry

        lax.fori_loop(0, NQ, quad, 0)
        plsc.subcore_barrier()
        for k in range(RC):
            base = rb + k * 128
            pltpu.sync_copy(agg_sp.at[pl.ds(base, 128)], msg[0])
            bias = AZ2 if final else Z2
            pltpu.sync_copy(bias.at[pl.ds(gb + k * 128, 128)], msg[1])
            scale = ci_h if final else cc_h
            pltpu.sync_copy(scale.at[pl.ds(base, 128)], msg[2])
            lax.fori_loop(0, 128, combine_row, 0)
            if final:
                pltpu.sync_copy(msg[0], out.at[pl.ds(gb + k * 128, 128)])
            else:
                pltpu.sync_copy(msg[0], s_h.at[pl.ds(gb + k * 128, 128)])
                pltpu.sync_copy(zero_v, agg_sp.at[pl.ds(base, 128)])
        if not final:
            plsc.subcore_barrier()


def _diffuse(S2, Z2, AZ2, cc1, ci1, srcp2, dstp, zer_h):
    mesh = plsc.VectorSubcoreMesh(core_axis_name="c", subcore_axis_name="s")
    out, _ = pl.kernel(
        _diff_body,
        out_type=(jax.ShapeDtypeStruct((2 * NP, DH), f32),
                  jax.ShapeDtypeStruct((2 * NP, DH), f32)),
        mesh=mesh,
        compiler_params=pltpu.CompilerParams(needs_layout_passes=False,
                                             use_tc_tiling_on_sc=False),
        scratch_types=[
            pltpu.VMEM((ECH, 128), i32),
            pltpu.VMEM((ECH, 128), i32),
            pltpu.VMEM((128, DH), f32),
            pltpu.VMEM((128, DH), f32),
            pltpu.VMEM((128, DH), f32),
            pltpu.VMEM((128, DH), f32),
            pltpu.VMEM((128, DH), f32),
            pltpu.VMEM_SHARED((NP, DH), f32),
            pltpu.SemaphoreType.DMA,
            pltpu.SemaphoreType.DMA,
            pltpu.SemaphoreType.DMA,
            pltpu.SemaphoreType.DMA,
        ],
    )(S2, Z2, AZ2, cc1, ci1, srcp2, dstp, zer_h)
    return out


def kernel(edge_index, feats, W1, b1, W2, b2):
    featsp = jnp.pad(feats, ((0, NP - N), (0, 0)))
    padidx = (N + (jnp.arange(EP - E, dtype=i32) % (NP - N))).astype(i32)
    srcp = jnp.concatenate([edge_index[0], padidx]).reshape(EP // 128, 128)
    dstp = jnp.concatenate([edge_index[1], padidx]).reshape(EP // 128, 128)
    ones_h = jnp.ones((128, 16), f32)
    zer16 = jnp.zeros((RT, 16), f32)
    zer_h = jnp.zeros((128, DH), f32)

    edf = jnp.concatenate([srcp, dstp])
    degs = _degrees(edf, ones_h, zer16)
    h1p, sA, sB, zA, zB, aA, aB, cc64, ci64 = _mlp(
        featsp, W1, b1.reshape(1, D), W2, b2.reshape(1, D),
        degs[:NP], degs[NP:])

    if False:
        no = lax.rsqrt(jnp.maximum(degs[:N, 0], 1.0))
        ni = lax.rsqrt(jnp.maximum(degs[NP:NP + N, 0], 1.0))
        h0 = jnp.concatenate([sA, sB], axis=1)[:N] / no[:, None]
        h = h0
        src = edge_index[0]
        dst = edge_index[1]
        for _ in range(K):
            hs = h * no[:, None]
            agg = jax.ops.segment_sum(jnp.take(hs, src, axis=0), dst,
                                      num_segments=N)
            h = (1.0 - ALPHA) * agg * ni[:, None] + ALPHA * h0
        return (h1p[:N], h)

    S2 = jnp.concatenate([sA, sB], axis=0)
    Z2 = jnp.concatenate([zA, zB], axis=0)
    AZ2 = jnp.concatenate([aA, aB], axis=0)
    srcp2 = jnp.concatenate([srcp, srcp + NP])
    out = _diffuse(S2, Z2, AZ2, cc64, ci64, srcp2, dstp, zer_h)
    h = jnp.concatenate([out[:N], out[NP:NP + N]], axis=1)
    return (h1p[:N], h)
